# Initial kernel scaffold; baseline (speedup 1.0000x reference)
import jax
import jax.numpy as jnp
from jax.experimental import pallas as pl


def kernel(x, grid0, grid1, grid2):
    raise NotImplementedError("write your pallas kernel here")


# speedup vs baseline: 1.9670x; 1.9670x over previous
import functools
import jax
import jax.numpy as jnp
from jax import lax
from jax.experimental import pallas as pl
from jax.experimental.pallas import tpu as pltpu
from jax.experimental.pallas import tpu_sc as plsc

B = 262144
C = 16
SIZES = (64, 96, 128)
NC = 2
NS = 16
NW = NC * NS
PPW = B // NW
CH = 128
NCHUNK = PPW // CH
L = 16
NG = CH // L


def _tec_kernel(xt, t0, t1, t2, out, coords_v, idx_v, w_v, rows_v, acc_v, sem):
    tables = (t0, t1, t2)
    wid = lax.axis_index("s") * NC + lax.axis_index("c")
    wbase = wid * PPW

    def chunk_body(ch, carry):
        base = wbase + ch * CH
        pltpu.sync_copy(xt.at[:, pl.ds(base, CH)], coords_v)

        for g, table in enumerate(tables):
            s = SIZES[g]
            scale = 0.5 * (s - 1)
            s2 = s * s
            offs = (0, 1, s, s + 1, s2, s2 + 1, s2 + s, s2 + s + 1)

            for i in range(NG):
                sl = pl.ds(i * L, L)
                gx = coords_v[0, sl]
                gy = coords_v[1, sl]
                gz = coords_v[2, sl]
                fx = gx * scale + scale
                fy = gy * scale + scale
                fz = gz * scale + scale
                x0 = jnp.minimum(jnp.maximum(fx.astype(jnp.int32), 0), s - 2)
                y0 = jnp.minimum(jnp.maximum(fy.astype(jnp.int32), 0), s - 2)
                z0 = jnp.minimum(jnp.maximum(fz.astype(jnp.int32), 0), s - 2)
                x0f = x0.astype(jnp.float32)
                y0f = y0.astype(jnp.float32)
                z0f = z0.astype(jnp.float32)
                wx1 = fx - x0f
                wy1 = fy - y0f
                wz1 = fz - z0f
                wx0 = 1.0 - wx1
                wy0 = 1.0 - wy1
                wz0 = 1.0 - wz1
                ibase = (z0 * s + y0) * s + x0
                a00 = wz0 * wy0
                a01 = wz0 * wy1
                a10 = wz1 * wy0
                a11 = wz1 * wy1
                ws = (a00 * wx0, a00 * wx1, a01 * wx0, a01 * wx1,
                      a10 * wx0, a10 * wx1, a11 * wx0, a11 * wx1)
                for k in range(8):
                    idx_v[k, sl] = ibase + offs[k]
                    w_v[k, sl] = ws[k]

            copies = [
                pltpu.async_copy(table.at[idx_v.at[k]], rows_v.at[k], sem)
                for k in range(8)
            ]
            for cp in copies:
                cp.wait()

            def acc_body(gi, carry2):
                off = gi * L
                sl = pl.ds(off, L)
                wr = [w_v[k, sl] for k in range(8)]
                for j in range(L):
                    p = off + j
                    acc = rows_v[0, p, :] * wr[0][j]
                    for k in range(1, 8):
                        acc = acc + rows_v[k, p, :] * wr[k][j]
                    acc_v[pl.ds(p * (3 * C) + g * C, C)] = acc
                return carry2

            lax.fori_loop(0, NG, acc_body, 0)

        pltpu.sync_copy(acc_v, out.at[pl.ds(base * (3 * C), CH * 3 * C)])
        return carry

    lax.fori_loop(0, NCHUNK, chunk_body, 0)


@jax.jit
def kernel(x, grid0, grid1, grid2):
    xt = x.T
    tables = [
        g.reshape(C, s * s * s).T
        for g, s in zip((grid0, grid1, grid2), SIZES)
    ]
    mesh = plsc.VectorSubcoreMesh(core_axis_name="c", subcore_axis_name="s")
    run = pl.kernel(
        _tec_kernel,
        out_type=jax.ShapeDtypeStruct((B * 3 * C,), jnp.float32),
        mesh=mesh,
        scratch_types=[
            pltpu.VMEM((3, CH), jnp.float32),
            pltpu.VMEM((8, CH), jnp.int32),
            pltpu.VMEM((8, CH), jnp.float32),
            pltpu.VMEM((8, CH, C), jnp.float32),
            pltpu.VMEM((CH * 3 * C,), jnp.float32),
            pltpu.SemaphoreType.DMA,
        ],
        compiler_params=pltpu.CompilerParams(use_tc_tiling_on_sc=False),
    )
    return run(xt, *tables).reshape(B, 3 * C)
